# baseline (speedup 1.0000x reference)
import functools

import jax
import jax.numpy as jnp
from jax.experimental import pallas as pl
from jax.experimental.pallas import tpu as pltpu

N = 4096
F_IN = 256
H = 256
C = 16
ORDERS = 3
NORM_LAYERS = 2
ALPHA = 0.1
BETA = 0.1
GAMMA = 0.5
DELTA = 0.5

COE = 1.0 / (ALPHA + BETA)
COE1 = 1.0 - GAMMA
COE2 = 1.0 / COE1

BM = 512
NB = N // BM
BMLP = 2048
NMLP = N // BMLP

bf16 = jnp.bfloat16
f32 = jnp.float32


def _bfdot(a, b, dims):
    return jax.lax.dot_general(a.astype(bf16), b.astype(bf16),
                               (dims, ((), ())),
                               preferred_element_type=f32)


def _eye():
    rows = jax.lax.broadcasted_iota(jnp.int32, (C, C), 0)
    cols = jax.lax.broadcasted_iota(jnp.int32, (C, C), 1)
    return jnp.where(rows == cols, 1.0, 0.0).astype(f32)


def _mlp_gram_kernel(x_ref, xa_ref, w1_ref, b1_ref, b3_ref, w2_ref, b2_ref,
                     xoT_ref, res1_ref, m_ref, acc_ref):
    i = pl.program_id(0)
    xX = _bfdot(x_ref[...], w1_ref[...], ((1,), (1,))) + b1_ref[...]
    xA = xa_ref[...] + b3_ref[...]
    h = jnp.maximum(DELTA * xX + (1.0 - DELTA) * xA, 0.0)
    xo = _bfdot(h, w2_ref[...], ((1,), (1,))) + b2_ref[...]
    xoT = xo.T
    xoT_ref[...] = xoT
    part = _bfdot(xoT, xoT, ((1,), (1,)))

    @pl.when(i == 0)
    def _():
        acc_ref[...] = part

    @pl.when(i > 0)
    def _():
        acc_ref[...] = acc_ref[...] + part

    @pl.when(i == NMLP - 1)
    def _():
        res1 = acc_ref[...]
        res1_ref[...] = res1
        m_ref[...] = (COE2 * COE2) * _eye() + COE * res1


def _prep_body(inv, res1, curT):
    r = _bfdot(inv, res1, ((1,), (0,)))
    xrT = _bfdot(r, curT, ((0,), (0,)))
    resT = (COE1 * COE) * curT - (COE1 * COE * COE) * xrT
    tmp = _bfdot(curT, resT, ((1,), (1,)))
    return resT, tmp


def _t1_cache_kernel(adj_ref, inv_ref, res1_ref, curT_ref, t1T_ref,
                     adjbf_ref, tmp_ref, resT_ref):
    i = pl.program_id(0)

    @pl.when(i == 0)
    def _():
        resT, tmp = _prep_body(inv_ref[...], res1_ref[...], curT_ref[...])
        resT_ref[...] = resT
        tmp_ref[...] = tmp

    a_bf = adj_ref[...].astype(bf16)
    adjbf_ref[...] = a_bf
    t1T_ref[...] = jax.lax.dot_general(resT_ref[...].astype(bf16), a_bf,
                                       ((((1,), (1,))), ((), ())),
                                       preferred_element_type=f32)


def _combine_body(t1T_blk, t2T_blk, t3T, curT_blk, h0T_blk, tmp, ow_ref):
    sT = t1T_blk * ow_ref[0, 0] + t2T_blk * ow_ref[1, 0] + t3T * ow_ref[2, 0]
    xtT = _bfdot(tmp, curT_blk, ((0,), (0,)))
    h0tT = _bfdot(tmp, h0T_blk, ((0,), (0,)))
    return (COE1 * xtT + BETA * sT - (GAMMA * COE1) * h0tT
            + GAMMA * h0T_blk)


def _layer1_tail_kernel(adjbf_ref, t1T_ref, curT_ref, tmp_ref, ow_ref,
                        out_ref, t2T_ref):
    s = pl.program_id(0)
    i = pl.program_id(1)
    a_bf = adjbf_ref[...]

    @pl.when(s == 0)
    def _():
        t2T_ref[:, pl.ds(i * BM, BM)] = jax.lax.dot_general(
            t1T_ref[...].astype(bf16), a_bf, ((((1,), (1,))), ((), ())),
            preferred_element_type=f32)

    @pl.when(s == 1)
    def _():
        t3T = jax.lax.dot_general(t2T_ref[...].astype(bf16), a_bf,
                                  ((((1,), (1,))), ((), ())),
                                  preferred_element_type=f32)
        blk = pl.ds(i * BM, BM)
        out_ref[...] = _combine_body(t1T_ref[:, blk], t2T_ref[:, blk], t3T,
                                     curT_ref[:, blk], curT_ref[:, blk],
                                     tmp_ref[...], ow_ref)


def _layer2_kernel(adjbf_ref, inv_ref, res1_ref, curT_ref, h0T_ref, ow_ref,
                   out_ref, resT_ref, tmp_ref, t1T_ref, t2T_ref):
    s = pl.program_id(0)
    i = pl.program_id(1)
    a_bf = adjbf_ref[...]

    @pl.when((s == 0) & (i == 0))
    def _():
        resT, tmp = _prep_body(inv_ref[...], res1_ref[...], curT_ref[...])
        resT_ref[...] = resT
        tmp_ref[...] = tmp

    @pl.when(s == 0)
    def _():
        t1T_ref[:, pl.ds(i * BM, BM)] = jax.lax.dot_general(
            resT_ref[...].astype(bf16), a_bf, ((((1,), (1,))), ((), ())),
            preferred_element_type=f32)

    @pl.when(s == 1)
    def _():
        t2T_ref[:, pl.ds(i * BM, BM)] = jax.lax.dot_general(
            t1T_ref[...].astype(bf16), a_bf, ((((1,), (1,))), ((), ())),
            preferred_element_type=f32)

    @pl.when(s == 2)
    def _():
        t3T = jax.lax.dot_general(t2T_ref[...].astype(bf16), a_bf,
                                  ((((1,), (1,))), ((), ())),
                                  preferred_element_type=f32)
        blk = pl.ds(i * BM, BM)
        cT = _combine_body(t1T_ref[:, blk], t2T_ref[:, blk], t3T,
                           curT_ref[:, blk], h0T_ref[:, blk], tmp_ref[...],
                           ow_ref)
        m = jnp.max(cT, axis=0, keepdims=True)
        shifted = cT - m
        lse = jnp.log(jnp.sum(jnp.exp(shifted), axis=0, keepdims=True))
        out_ref[...] = (shifted - lse).T


def _gram_kernel(curT_ref, res1_ref, m_ref):
    curT = curT_ref[...]
    res1 = _bfdot(curT, curT, ((1,), (1,)))
    res1_ref[...] = res1
    m_ref[...] = (COE2 * COE2) * _eye() + COE * res1


def kernel(x, adj, y_clean, y_unknown, if_lp, W1, b1, W2, b2, W3, b3,
           orders_weight):
    del y_clean, y_unknown, if_lp
    xa = adj @ W3.T

    b1r = b1.reshape(1, H)
    b3r = b3.reshape(1, H)
    b2r = b2.reshape(1, C)

    row_blk = lambda i: (i, 0)
    col_blk = lambda i: (0, i)
    full2 = lambda i: (0, 0)
    full0 = lambda: (0, 0)
    srow_blk = lambda s, i: (i, 0)
    sfull = lambda s, i: (0, 0)
    last1_col = lambda s, i: (0, jnp.where(s == 1, i, 0))
    last2_row = lambda s, i: (jnp.where(s == 2, i, 0), 0)

    xoT, res1_1, m1 = pl.pallas_call(
        _mlp_gram_kernel,
        grid=(NMLP,),
        in_specs=[
            pl.BlockSpec((BMLP, F_IN), row_blk),
            pl.BlockSpec((BMLP, H), row_blk),
            pl.BlockSpec((H, F_IN), full2),
            pl.BlockSpec((1, H), full2),
            pl.BlockSpec((1, H), full2),
            pl.BlockSpec((C, H), full2),
            pl.BlockSpec((1, C), full2),
        ],
        out_specs=[
            pl.BlockSpec((C, BMLP), col_blk),
            pl.BlockSpec((C, C), full2),
            pl.BlockSpec((C, C), full2),
        ],
        out_shape=[
            jax.ShapeDtypeStruct((C, N), f32),
            jax.ShapeDtypeStruct((C, C), f32),
            jax.ShapeDtypeStruct((C, C), f32),
        ],
        scratch_shapes=[pltpu.VMEM((C, C), f32)],
    )(x, xa, W1, b1r, b3r, W2, b2r)

    inv1 = jnp.linalg.pinv(m1)

    t1T, adjbf, tmp1 = pl.pallas_call(
        _t1_cache_kernel,
        grid=(NB,),
        in_specs=[
            pl.BlockSpec((BM, N), row_blk),
            pl.BlockSpec((C, C), full2),
            pl.BlockSpec((C, C), full2),
            pl.BlockSpec((C, N), full2),
        ],
        out_specs=[
            pl.BlockSpec((C, BM), col_blk),
            pl.BlockSpec((BM, N), row_blk),
            pl.BlockSpec((C, C), full2),
        ],
        out_shape=[
            jax.ShapeDtypeStruct((C, N), f32),
            jax.ShapeDtypeStruct((N, N), bf16),
            jax.ShapeDtypeStruct((C, C), f32),
        ],
        scratch_shapes=[pltpu.VMEM((C, N), f32)],
    )(adj, inv1, res1_1, xoT)

    curT2 = pl.pallas_call(
        _layer1_tail_kernel,
        grid=(2, NB),
        in_specs=[
            pl.BlockSpec((BM, N), srow_blk),
            pl.BlockSpec((C, N), sfull),
            pl.BlockSpec((C, N), sfull),
            pl.BlockSpec((C, C), sfull),
            pl.BlockSpec((3, 1), sfull),
        ],
        out_specs=pl.BlockSpec((C, BM), last1_col),
        out_shape=jax.ShapeDtypeStruct((C, N), f32),
        scratch_shapes=[pltpu.VMEM((C, N), f32)],
    )(adjbf, t1T, xoT, tmp1, orders_weight)

    res1_2, m2 = pl.pallas_call(
        _gram_kernel,
        in_specs=[pl.BlockSpec((C, N), full0)],
        out_specs=[
            pl.BlockSpec((C, C), full0),
            pl.BlockSpec((C, C), full0),
        ],
        out_shape=[
            jax.ShapeDtypeStruct((C, C), f32),
            jax.ShapeDtypeStruct((C, C), f32),
        ],
    )(curT2)

    inv2 = jnp.linalg.pinv(m2)

    out = pl.pallas_call(
        _layer2_kernel,
        grid=(3, NB),
        in_specs=[
            pl.BlockSpec((BM, N), srow_blk),
            pl.BlockSpec((C, C), sfull),
            pl.BlockSpec((C, C), sfull),
            pl.BlockSpec((C, N), sfull),
            pl.BlockSpec((C, N), sfull),
            pl.BlockSpec((3, 1), sfull),
        ],
        out_specs=pl.BlockSpec((BM, C), last2_row),
        out_shape=jax.ShapeDtypeStruct((N, C), f32),
        scratch_shapes=[
            pltpu.VMEM((C, N), f32),
            pltpu.VMEM((C, C), f32),
            pltpu.VMEM((C, N), f32),
            pltpu.VMEM((C, N), f32),
        ],
    )(adjbf, inv2, res1_2, curT2, xoT, orders_weight)
    return out


# speedup vs baseline: 1.0290x; 1.0290x over previous
import functools

import jax
import jax.numpy as jnp
from jax.experimental import pallas as pl
from jax.experimental.pallas import tpu as pltpu

N = 4096
F_IN = 256
H = 256
C = 16
ORDERS = 3
NORM_LAYERS = 2
ALPHA = 0.1
BETA = 0.1
GAMMA = 0.5
DELTA = 0.5

COE = 1.0 / (ALPHA + BETA)
COE1 = 1.0 - GAMMA
COE2 = 1.0 / COE1

BM = 1024
NB = N // BM
BMD = 2048
NBD = N // BMD
BMLP = 2048
NMLP = N // BMLP

bf16 = jnp.bfloat16
f32 = jnp.float32


def _bfdot(a, b, dims):
    return jax.lax.dot_general(a.astype(bf16), b.astype(bf16),
                               (dims, ((), ())),
                               preferred_element_type=f32)


def _eye():
    rows = jax.lax.broadcasted_iota(jnp.int32, (C, C), 0)
    cols = jax.lax.broadcasted_iota(jnp.int32, (C, C), 1)
    return jnp.where(rows == cols, 1.0, 0.0).astype(f32)


def _mlp_gram_kernel(x_ref, xa_ref, w1_ref, b1_ref, b3_ref, w2_ref, b2_ref,
                     xoT_ref, res1_ref, m_ref, acc_ref):
    i = pl.program_id(0)
    xX = _bfdot(x_ref[...], w1_ref[...], ((1,), (1,))) + b1_ref[...]
    xA = xa_ref[...] + b3_ref[...]
    h = jnp.maximum(DELTA * xX + (1.0 - DELTA) * xA, 0.0)
    xo = _bfdot(h, w2_ref[...], ((1,), (1,))) + b2_ref[...]
    xoT = xo.T
    xoT_ref[...] = xoT
    part = _bfdot(xoT, xoT, ((1,), (1,)))

    @pl.when(i == 0)
    def _():
        acc_ref[...] = part

    @pl.when(i > 0)
    def _():
        acc_ref[...] = acc_ref[...] + part

    @pl.when(i == NMLP - 1)
    def _():
        res1 = acc_ref[...]
        res1_ref[...] = res1
        m_ref[...] = (COE2 * COE2) * _eye() + COE * res1


def _prep_body(inv, res1, curT):
    r = _bfdot(inv, res1, ((1,), (0,)))
    xrT = _bfdot(r, curT, ((0,), (0,)))
    resT = (COE1 * COE) * curT - (COE1 * COE * COE) * xrT
    tmp = _bfdot(curT, resT, ((1,), (1,)))
    return resT, tmp


def _t1_cache_kernel(adj_ref, inv_ref, res1_ref, curT_ref, t1T_ref,
                     adjbf_ref, tmp_ref, resT_ref):
    i = pl.program_id(0)

    @pl.when(i == 0)
    def _():
        resT, tmp = _prep_body(inv_ref[...], res1_ref[...], curT_ref[...])
        resT_ref[...] = resT
        tmp_ref[...] = tmp

    a_bf = adj_ref[...].astype(bf16)
    adjbf_ref[...] = a_bf
    t1T_ref[...] = jax.lax.dot_general(resT_ref[...].astype(bf16), a_bf,
                                       ((((1,), (1,))), ((), ())),
                                       preferred_element_type=f32)


def _combine_body(t1T_blk, t2T_blk, t3T, curT_blk, h0T_blk, tmp, ow_ref):
    sT = t1T_blk * ow_ref[0, 0] + t2T_blk * ow_ref[1, 0] + t3T * ow_ref[2, 0]
    xtT = _bfdot(tmp, curT_blk, ((0,), (0,)))
    h0tT = _bfdot(tmp, h0T_blk, ((0,), (0,)))
    return (COE1 * xtT + BETA * sT - (GAMMA * COE1) * h0tT
            + GAMMA * h0T_blk)


def _layer1_tail_kernel(adjbf_ref, t1T_ref, curT_ref, tmp_ref, ow_ref,
                        out_ref, t2T_ref):
    s = pl.program_id(0)
    i = pl.program_id(1)
    a_bf = adjbf_ref[...]

    @pl.when(s == 0)
    def _():
        t2T_ref[:, pl.ds(i * BMD, BMD)] = jax.lax.dot_general(
            t1T_ref[...].astype(bf16), a_bf, ((((1,), (1,))), ((), ())),
            preferred_element_type=f32)

    @pl.when(s == 1)
    def _():
        t3T = jax.lax.dot_general(t2T_ref[...].astype(bf16), a_bf,
                                  ((((1,), (1,))), ((), ())),
                                  preferred_element_type=f32)
        blk = pl.ds(i * BMD, BMD)
        out_ref[...] = _combine_body(t1T_ref[:, blk], t2T_ref[:, blk], t3T,
                                     curT_ref[:, blk], curT_ref[:, blk],
                                     tmp_ref[...], ow_ref)


def _layer2_kernel(adjbf_ref, inv_ref, res1_ref, curT_ref, h0T_ref, ow_ref,
                   out_ref, resT_ref, tmp_ref, t1T_ref, t2T_ref):
    s = pl.program_id(0)
    i = pl.program_id(1)
    a_bf = adjbf_ref[...]

    @pl.when((s == 0) & (i == 0))
    def _():
        resT, tmp = _prep_body(inv_ref[...], res1_ref[...], curT_ref[...])
        resT_ref[...] = resT
        tmp_ref[...] = tmp

    @pl.when(s == 0)
    def _():
        t1T_ref[:, pl.ds(i * BMD, BMD)] = jax.lax.dot_general(
            resT_ref[...].astype(bf16), a_bf, ((((1,), (1,))), ((), ())),
            preferred_element_type=f32)

    @pl.when(s == 1)
    def _():
        t2T_ref[:, pl.ds(i * BMD, BMD)] = jax.lax.dot_general(
            t1T_ref[...].astype(bf16), a_bf, ((((1,), (1,))), ((), ())),
            preferred_element_type=f32)

    @pl.when(s == 2)
    def _():
        t3T = jax.lax.dot_general(t2T_ref[...].astype(bf16), a_bf,
                                  ((((1,), (1,))), ((), ())),
                                  preferred_element_type=f32)
        blk = pl.ds(i * BMD, BMD)
        cT = _combine_body(t1T_ref[:, blk], t2T_ref[:, blk], t3T,
                           curT_ref[:, blk], h0T_ref[:, blk], tmp_ref[...],
                           ow_ref)
        m = jnp.max(cT, axis=0, keepdims=True)
        shifted = cT - m
        lse = jnp.log(jnp.sum(jnp.exp(shifted), axis=0, keepdims=True))
        out_ref[...] = (shifted - lse).T


def _gram_kernel(curT_ref, res1_ref, m_ref):
    curT = curT_ref[...]
    res1 = _bfdot(curT, curT, ((1,), (1,)))
    res1_ref[...] = res1
    m_ref[...] = (COE2 * COE2) * _eye() + COE * res1


def kernel(x, adj, y_clean, y_unknown, if_lp, W1, b1, W2, b2, W3, b3,
           orders_weight):
    del y_clean, y_unknown, if_lp
    xa = adj @ W3.T

    b1r = b1.reshape(1, H)
    b3r = b3.reshape(1, H)
    b2r = b2.reshape(1, C)

    row_blk = lambda i: (i, 0)
    col_blk = lambda i: (0, i)
    full2 = lambda i: (0, 0)
    full0 = lambda: (0, 0)
    srow_blk = lambda s, i: (i, 0)
    sfull = lambda s, i: (0, 0)
    last1_col = lambda s, i: (0, jnp.where(s == 1, i, 0))
    last2_row = lambda s, i: (jnp.where(s == 2, i, 0), 0)

    xoT, res1_1, m1 = pl.pallas_call(
        _mlp_gram_kernel,
        grid=(NMLP,),
        in_specs=[
            pl.BlockSpec((BMLP, F_IN), row_blk),
            pl.BlockSpec((BMLP, H), row_blk),
            pl.BlockSpec((H, F_IN), full2),
            pl.BlockSpec((1, H), full2),
            pl.BlockSpec((1, H), full2),
            pl.BlockSpec((C, H), full2),
            pl.BlockSpec((1, C), full2),
        ],
        out_specs=[
            pl.BlockSpec((C, BMLP), col_blk),
            pl.BlockSpec((C, C), full2),
            pl.BlockSpec((C, C), full2),
        ],
        out_shape=[
            jax.ShapeDtypeStruct((C, N), f32),
            jax.ShapeDtypeStruct((C, C), f32),
            jax.ShapeDtypeStruct((C, C), f32),
        ],
        scratch_shapes=[pltpu.VMEM((C, C), f32)],
    )(x, xa, W1, b1r, b3r, W2, b2r)

    inv1 = jnp.linalg.pinv(m1)

    t1T, adjbf, tmp1 = pl.pallas_call(
        _t1_cache_kernel,
        grid=(NB,),
        in_specs=[
            pl.BlockSpec((BM, N), row_blk),
            pl.BlockSpec((C, C), full2),
            pl.BlockSpec((C, C), full2),
            pl.BlockSpec((C, N), full2),
        ],
        out_specs=[
            pl.BlockSpec((C, BM), col_blk),
            pl.BlockSpec((BM, N), row_blk),
            pl.BlockSpec((C, C), full2),
        ],
        out_shape=[
            jax.ShapeDtypeStruct((C, N), f32),
            jax.ShapeDtypeStruct((N, N), bf16),
            jax.ShapeDtypeStruct((C, C), f32),
        ],
        scratch_shapes=[pltpu.VMEM((C, N), f32)],
    )(adj, inv1, res1_1, xoT)

    curT2 = pl.pallas_call(
        _layer1_tail_kernel,
        grid=(2, NBD),
        in_specs=[
            pl.BlockSpec((BMD, N), srow_blk),
            pl.BlockSpec((C, N), sfull),
            pl.BlockSpec((C, N), sfull),
            pl.BlockSpec((C, C), sfull),
            pl.BlockSpec((3, 1), sfull),
        ],
        out_specs=pl.BlockSpec((C, BMD), last1_col),
        out_shape=jax.ShapeDtypeStruct((C, N), f32),
        scratch_shapes=[pltpu.VMEM((C, N), f32)],
    )(adjbf, t1T, xoT, tmp1, orders_weight)

    res1_2, m2 = pl.pallas_call(
        _gram_kernel,
        in_specs=[pl.BlockSpec((C, N), full0)],
        out_specs=[
            pl.BlockSpec((C, C), full0),
            pl.BlockSpec((C, C), full0),
        ],
        out_shape=[
            jax.ShapeDtypeStruct((C, C), f32),
            jax.ShapeDtypeStruct((C, C), f32),
        ],
    )(curT2)

    inv2 = jnp.linalg.pinv(m2)

    out = pl.pallas_call(
        _layer2_kernel,
        grid=(3, NBD),
        in_specs=[
            pl.BlockSpec((BMD, N), srow_blk),
            pl.BlockSpec((C, C), sfull),
            pl.BlockSpec((C, C), sfull),
            pl.BlockSpec((C, N), sfull),
            pl.BlockSpec((C, N), sfull),
            pl.BlockSpec((3, 1), sfull),
        ],
        out_specs=pl.BlockSpec((BMD, C), last2_row),
        out_shape=jax.ShapeDtypeStruct((N, C), f32),
        scratch_shapes=[
            pltpu.VMEM((C, N), f32),
            pltpu.VMEM((C, C), f32),
            pltpu.VMEM((C, N), f32),
            pltpu.VMEM((C, N), f32),
        ],
    )(adjbf, inv2, res1_2, curT2, xoT, orders_weight)
    return out
